# baseline (speedup 1.0000x reference)
import functools

import jax
import jax.numpy as jnp
from jax import lax
from jax.experimental import pallas as pl
from jax.experimental.pallas import tpu as pltpu
from jax.experimental.pallas import tpu_sc as plsc

NC = 2
NS = 16
NW = NC * NS

LANE = 128
CHUNK = 128
RING = 4
UNROLL = 2


def _rowsum_tc(table):
    v, d = table.shape
    t_t = jnp.swapaxes(table, 0, 1)
    cblk = 32768
    grid = (v + cblk - 1) // cblk

    def body(t_ref, o_ref):
        o_ref[...] = jnp.sum(t_ref[...], axis=0)

    return pl.pallas_call(
        body,
        grid=(grid,),
        in_specs=[pl.BlockSpec((d, cblk), lambda i: (0, i))],
        out_specs=pl.BlockSpec((cblk,), lambda i: (i,)),
        out_shape=jax.ShapeDtypeStruct((v,), jnp.float32),
    )(t_t)


def _split_tc(indices, s_a, s_b):
    b, s = indices.shape
    rblk = 512
    assert b % rblk == 0

    def body(i_ref, a_ref, b_ref):
        x = i_ref[...]
        a_ref[...] = x[:, :s_a]
        b_ref[...] = jnp.concatenate(
            [x[:, s_a:s], jnp.zeros((rblk, LANE - s_b), jnp.int32)], axis=1)

    return pl.pallas_call(
        body,
        grid=(b // rblk,),
        in_specs=[pl.BlockSpec((rblk, s), lambda i: (i, 0))],
        out_specs=[pl.BlockSpec((rblk, LANE), lambda i: (i, 0)),
                   pl.BlockSpec((rblk, LANE), lambda i: (i, 0))],
        out_shape=[jax.ShapeDtypeStruct((b, LANE), jnp.int32),
                   jax.ShapeDtypeStruct((b, LANE), jnp.int32)],
    )(indices)


def _gather_sc(idx_a, idx_b, rsflat):
    n_rows = idx_a.shape[0]
    assert (n_rows * LANE) % (NW * CHUNK) == 0
    per_w = n_rows * LANE // NW
    n_ch = per_w // CHUNK
    assert (n_ch - RING) % UNROLL == 0
    ia_flat = idx_a.reshape(n_rows * LANE)
    ib_flat = idx_b.reshape(n_rows * LANE)

    mesh = plsc.VectorSubcoreMesh(core_axis_name="c", subcore_axis_name="s")

    @functools.partial(
        pl.kernel,
        mesh=mesh,
        out_type=[jax.ShapeDtypeStruct((n_rows * LANE,), jnp.float32),
                  jax.ShapeDtypeStruct((n_rows * LANE,), jnp.float32)],
        scratch_types=[
            pltpu.VMEM((per_w,), jnp.int32),
            pltpu.VMEM((per_w,), jnp.int32),
            pltpu.VMEM((per_w,), jnp.float32),
            pltpu.VMEM((per_w,), jnp.float32),
            pltpu.SemaphoreType.DMA,
        ],
    )
    def k(ia_hbm, ib_hbm, rs_hbm, oa_hbm, ob_hbm, ia_v, ib_v, va_v, vb_v, sem):
        wid = lax.axis_index("s") * NC + lax.axis_index("c")
        base = wid * per_w
        pltpu.sync_copy(ia_hbm.at[pl.ds(base, per_w)], ia_v)
        pltpu.sync_copy(ib_hbm.at[pl.ds(base, per_w)], ib_v)

        def mk(idx_v, val_v, c):
            return pltpu.make_async_copy(
                rs_hbm.at[idx_v.at[pl.ds(c * CHUNK, CHUNK)]],
                val_v.at[pl.ds(c * CHUNK, CHUNK)], sem)

        def ring(idx_v, val_v):
            def prime(c, carry):
                mk(idx_v, val_v, c).start()
                return carry
            lax.fori_loop(0, RING, prime, 0)

            def step(i, carry):
                c = i * UNROLL
                for u in range(UNROLL):
                    mk(idx_v, val_v, c + u + RING).start()
                    mk(idx_v, val_v, c + u).wait()
                return carry
            lax.fori_loop(0, (n_ch - RING) // UNROLL, step, 0)

            def drain(c, carry):
                mk(idx_v, val_v, c).wait()
                return carry
            lax.fori_loop(n_ch - RING, n_ch, drain, 0)

        ring(ia_v, va_v)
        ring(ib_v, vb_v)

        pltpu.sync_copy(va_v, oa_hbm.at[pl.ds(base, per_w)])
        pltpu.sync_copy(vb_v, ob_hbm.at[pl.ds(base, per_w)])

    ga, gb = k(ia_flat, ib_flat, rsflat)
    return ga.reshape(n_rows, LANE), gb.reshape(n_rows, LANE)


def _stitch_mul_tc(g_a, g_b, weights, s_a, s_b):
    b, s = weights.shape
    rblk = 512

    def body(a_ref, b_ref, w_ref, o_ref):
        g = jnp.concatenate([a_ref[...], b_ref[:, :s_b]], axis=1)
        o_ref[...] = g * w_ref[...]

    return pl.pallas_call(
        body,
        grid=(b // rblk,),
        in_specs=[
            pl.BlockSpec((rblk, LANE), lambda i: (i, 0)),
            pl.BlockSpec((rblk, LANE), lambda i: (i, 0)),
            pl.BlockSpec((rblk, s), lambda i: (i, 0)),
        ],
        out_specs=pl.BlockSpec((rblk, s), lambda i: (i, 0)),
        out_shape=jax.ShapeDtypeStruct((b, s), jnp.float32),
    )(g_a, g_b, weights)


def kernel(indices, weights, table):
    b, s, n = indices.shape
    assert n == 1 and b % NW == 0
    s_a = LANE
    s_b = s - s_a
    rsflat = _rowsum_tc(table)
    idx_a, idx_b = _split_tc(indices.reshape(b, s).astype(jnp.int32), s_a, s_b)
    g_a, g_b = _gather_sc(idx_a, idx_b, rsflat)
    return _stitch_mul_tc(g_a, g_b, weights.reshape(b, s), s_a, s_b)


# speedup vs baseline: 9.0661x; 9.0661x over previous
import functools

import jax
import jax.numpy as jnp
from jax import lax
from jax.experimental import pallas as pl
from jax.experimental.pallas import tpu as pltpu
from jax.experimental.pallas import tpu_sc as plsc

NC = 2
NS = 16
NW = NC * NS

LANE = 128
RING = 32
UNROLL = 8


def _rowsum_tc(table):
    v, d = table.shape
    t_t = jnp.swapaxes(table, 0, 1)
    cblk = 32768
    grid = (v + cblk - 1) // cblk

    def body(t_ref, o_ref):
        o_ref[...] = jnp.sum(t_ref[...], axis=0)

    return pl.pallas_call(
        body,
        grid=(grid,),
        in_specs=[pl.BlockSpec((d, cblk), lambda i: (0, i))],
        out_specs=pl.BlockSpec((cblk,), lambda i: (i,)),
        out_shape=jax.ShapeDtypeStruct((v,), jnp.float32),
    )(t_t)


def _split_tc(indices, s_a, s_b):
    b, s = indices.shape
    rblk = 512
    assert b % rblk == 0

    def body(i_ref, a_ref, b_ref):
        x = i_ref[...]
        a_ref[...] = x[:, :s_a]
        b_ref[...] = jnp.concatenate(
            [x[:, s_a:s], jnp.zeros((rblk, LANE - s_b), jnp.int32)], axis=1)

    return pl.pallas_call(
        body,
        grid=(b // rblk,),
        in_specs=[pl.BlockSpec((rblk, s), lambda i: (i, 0))],
        out_specs=[pl.BlockSpec((rblk, LANE), lambda i: (i, 0)),
                   pl.BlockSpec((rblk, LANE), lambda i: (i, 0))],
        out_shape=[jax.ShapeDtypeStruct((b, LANE), jnp.int32),
                   jax.ShapeDtypeStruct((b, LANE), jnp.int32)],
    )(indices)


def _gather_sc(idx_a, idx_b, rsflat, s_b):
    n_rows = idx_a.shape[0]
    assert n_rows % NW == 0
    n_ch = n_rows // NW

    mesh = plsc.VectorSubcoreMesh(core_axis_name="c", subcore_axis_name="s")

    @functools.partial(
        pl.kernel,
        mesh=mesh,
        out_type=[jax.ShapeDtypeStruct((n_rows, LANE), jnp.float32),
                  jax.ShapeDtypeStruct((n_rows, LANE), jnp.float32)],
        scratch_types=[
            pltpu.VMEM((n_ch, LANE), jnp.int32),
            pltpu.VMEM((n_ch, LANE), jnp.int32),
            pltpu.VMEM((n_ch, LANE), jnp.float32),
            pltpu.VMEM((n_ch, LANE), jnp.float32),
            pltpu.SemaphoreType.DMA,
        ],
    )
    def k(ia_hbm, ib_hbm, rs_hbm, oa_hbm, ob_hbm, ia_v, ib_v, va_v, vb_v, sem):
        wid = lax.axis_index("s") * NC + lax.axis_index("c")
        row0 = wid * n_ch
        pltpu.sync_copy(ia_hbm.at[pl.ds(row0, n_ch)], ia_v)
        pltpu.sync_copy(ib_hbm.at[pl.ds(row0, n_ch)], ib_v)

        def start_a(c):
            pltpu.make_async_copy(
                rs_hbm.at[ia_v.at[c]], va_v.at[c], sem).start()

        def wait_a(c):
            pltpu.make_async_copy(
                rs_hbm.at[ia_v.at[c]], va_v.at[c], sem).wait()

        def start_b(c):
            pltpu.make_async_copy(
                rs_hbm.at[ib_v.at[c, pl.ds(0, s_b)]],
                vb_v.at[c, pl.ds(0, s_b)], sem).start()

        def wait_b(c):
            pltpu.make_async_copy(
                rs_hbm.at[ib_v.at[c, pl.ds(0, s_b)]],
                vb_v.at[c, pl.ds(0, s_b)], sem).wait()

        def ring(start, wait):
            assert (n_ch - RING) % UNROLL == 0

            def prime(c, carry):
                start(c)
                return carry
            lax.fori_loop(0, RING, prime, 0)

            def step(i, carry):
                c = i * UNROLL
                for u in range(UNROLL):
                    start(c + u + RING)
                    wait(c + u)
                return carry
            lax.fori_loop(0, (n_ch - RING) // UNROLL, step, 0)

            def drain(c, carry):
                wait(c)
                return carry
            lax.fori_loop(n_ch - RING, n_ch, drain, 0)

        ring(start_a, wait_a)
        ring(start_b, wait_b)

        pltpu.sync_copy(va_v, oa_hbm.at[pl.ds(row0, n_ch)])
        pltpu.sync_copy(vb_v, ob_hbm.at[pl.ds(row0, n_ch)])

    return k(idx_a, idx_b, rsflat)


def _stitch_mul_tc(g_a, g_b, weights, s_a, s_b):
    b, s = weights.shape
    rblk = 512

    def body(a_ref, b_ref, w_ref, o_ref):
        g = jnp.concatenate([a_ref[...], b_ref[:, :s_b]], axis=1)
        o_ref[...] = g * w_ref[...]

    return pl.pallas_call(
        body,
        grid=(b // rblk,),
        in_specs=[
            pl.BlockSpec((rblk, LANE), lambda i: (i, 0)),
            pl.BlockSpec((rblk, LANE), lambda i: (i, 0)),
            pl.BlockSpec((rblk, s), lambda i: (i, 0)),
        ],
        out_specs=pl.BlockSpec((rblk, s), lambda i: (i, 0)),
        out_shape=jax.ShapeDtypeStruct((b, s), jnp.float32),
    )(g_a, g_b, weights)


def kernel(indices, weights, table):
    b, s, n = indices.shape
    assert n == 1 and b % NW == 0
    s_a = LANE
    s_b = s - s_a
    rsflat = _rowsum_tc(table)
    idx_a, idx_b = _split_tc(indices.reshape(b, s).astype(jnp.int32), s_a, s_b)
    g_a, g_b = _gather_sc(idx_a, idx_b, rsflat, s_b)
    return _stitch_mul_tc(g_a, g_b, weights.reshape(b, s), s_a, s_b)


# speedup vs baseline: 9.6607x; 1.0656x over previous
import functools

import jax
import jax.numpy as jnp
from jax import lax
from jax.experimental import pallas as pl
from jax.experimental.pallas import tpu as pltpu
from jax.experimental.pallas import tpu_sc as plsc

NC = 2
NS = 16
NW = NC * NS

LANE = 128
RING = 64
UNROLL = 8


def _rowsum_tc(table):
    v, d = table.shape
    t_t = jnp.swapaxes(table, 0, 1)
    cblk = 65536
    grid = (v + cblk - 1) // cblk

    def body(t_ref, o_ref):
        o_ref[...] = jnp.sum(t_ref[...], axis=0)

    return pl.pallas_call(
        body,
        grid=(grid,),
        in_specs=[pl.BlockSpec((d, cblk), lambda i: (0, i))],
        out_specs=pl.BlockSpec((cblk,), lambda i: (i,)),
        out_shape=jax.ShapeDtypeStruct((v,), jnp.float32),
    )(t_t)


def _split_tc(indices, s_a, s_b):
    b, s = indices.shape
    rblk = 512
    assert b % rblk == 0

    def body(i_ref, a_ref, b_ref):
        x = i_ref[...]
        a_ref[...] = x[:, :s_a]
        b_ref[...] = jnp.concatenate(
            [x[:, s_a:s], jnp.zeros((rblk, LANE - s_b), jnp.int32)], axis=1)

    return pl.pallas_call(
        body,
        grid=(b // rblk,),
        in_specs=[pl.BlockSpec((rblk, s), lambda i: (i, 0))],
        out_specs=[pl.BlockSpec((rblk, LANE), lambda i: (i, 0)),
                   pl.BlockSpec((rblk, LANE), lambda i: (i, 0))],
        out_shape=[jax.ShapeDtypeStruct((b, LANE), jnp.int32),
                   jax.ShapeDtypeStruct((b, LANE), jnp.int32)],
    )(indices)


def _gather_sc(idx_a, idx_b, rsflat, s_b):
    n_rows = idx_a.shape[0]
    assert n_rows % NW == 0
    n_ch = n_rows // NW

    mesh = plsc.VectorSubcoreMesh(core_axis_name="c", subcore_axis_name="s")

    @functools.partial(
        pl.kernel,
        mesh=mesh,
        out_type=[jax.ShapeDtypeStruct((n_rows, LANE), jnp.float32),
                  jax.ShapeDtypeStruct((n_rows, LANE), jnp.float32)],
        scratch_types=[
            pltpu.VMEM((n_ch, LANE), jnp.int32),
            pltpu.VMEM((n_ch, LANE), jnp.int32),
            pltpu.VMEM((n_ch, LANE), jnp.float32),
            pltpu.VMEM((n_ch, LANE), jnp.float32),
            pltpu.SemaphoreType.DMA,
        ],
    )
    def k(ia_hbm, ib_hbm, rs_hbm, oa_hbm, ob_hbm, ia_v, ib_v, va_v, vb_v, sem):
        wid = lax.axis_index("s") * NC + lax.axis_index("c")
        row0 = wid * n_ch
        pltpu.sync_copy(ia_hbm.at[pl.ds(row0, n_ch)], ia_v)
        pltpu.sync_copy(ib_hbm.at[pl.ds(row0, n_ch)], ib_v)

        def start_a(c):
            pltpu.make_async_copy(
                rs_hbm.at[ia_v.at[c]], va_v.at[c], sem).start()

        def wait_a(c):
            pltpu.make_async_copy(
                rs_hbm.at[ia_v.at[c]], va_v.at[c], sem).wait()

        def start_b(c):
            pltpu.make_async_copy(
                rs_hbm.at[ib_v.at[c, pl.ds(0, s_b)]],
                vb_v.at[c, pl.ds(0, s_b)], sem).start()

        def wait_b(c):
            pltpu.make_async_copy(
                rs_hbm.at[ib_v.at[c, pl.ds(0, s_b)]],
                vb_v.at[c, pl.ds(0, s_b)], sem).wait()

        def ring(start, wait):
            assert (n_ch - RING) % UNROLL == 0

            def prime(c, carry):
                start(c)
                return carry
            lax.fori_loop(0, RING, prime, 0)

            def step(i, carry):
                c = i * UNROLL
                for u in range(UNROLL):
                    start(c + u + RING)
                    wait(c + u)
                return carry
            lax.fori_loop(0, (n_ch - RING) // UNROLL, step, 0)

            def drain(c, carry):
                wait(c)
                return carry
            lax.fori_loop(n_ch - RING, n_ch, drain, 0)

        ring(start_a, wait_a)
        ring(start_b, wait_b)

        pltpu.sync_copy(va_v, oa_hbm.at[pl.ds(row0, n_ch)])
        pltpu.sync_copy(vb_v, ob_hbm.at[pl.ds(row0, n_ch)])

    return k(idx_a, idx_b, rsflat)


def _stitch_mul_tc(g_a, g_b, weights, s_a, s_b):
    b, s = weights.shape
    rblk = 512

    def body(a_ref, b_ref, w_ref, o_ref):
        g = jnp.concatenate([a_ref[...], b_ref[:, :s_b]], axis=1)
        o_ref[...] = g * w_ref[...]

    return pl.pallas_call(
        body,
        grid=(b // rblk,),
        in_specs=[
            pl.BlockSpec((rblk, LANE), lambda i: (i, 0)),
            pl.BlockSpec((rblk, LANE), lambda i: (i, 0)),
            pl.BlockSpec((rblk, s), lambda i: (i, 0)),
        ],
        out_specs=pl.BlockSpec((rblk, s), lambda i: (i, 0)),
        out_shape=jax.ShapeDtypeStruct((b, s), jnp.float32),
    )(g_a, g_b, weights)


def kernel(indices, weights, table):
    b, s, n = indices.shape
    assert n == 1 and b % NW == 0
    s_a = LANE
    s_b = s - s_a
    rsflat = _rowsum_tc(table)
    idx_a, idx_b = _split_tc(indices.reshape(b, s).astype(jnp.int32), s_a, s_b)
    g_a, g_b = _gather_sc(idx_a, idx_b, rsflat, s_b)
    return _stitch_mul_tc(g_a, g_b, weights.reshape(b, s), s_a, s_b)


# speedup vs baseline: 9.8577x; 1.0204x over previous
import functools

import jax
import jax.numpy as jnp
from jax import lax
from jax.experimental import pallas as pl
from jax.experimental.pallas import tpu as pltpu
from jax.experimental.pallas import tpu_sc as plsc

NC = 2
NS = 16
NW = NC * NS

LANE = 128
RING = 96
UNROLL = 8


def _rowsum_tc(table):
    v, d = table.shape
    t_t = jnp.swapaxes(table, 0, 1)
    cblk = 65536
    grid = (v + cblk - 1) // cblk

    def body(t_ref, o_ref):
        o_ref[...] = jnp.sum(t_ref[...], axis=0)

    return pl.pallas_call(
        body,
        grid=(grid,),
        in_specs=[pl.BlockSpec((d, cblk), lambda i: (0, i))],
        out_specs=pl.BlockSpec((cblk,), lambda i: (i,)),
        out_shape=jax.ShapeDtypeStruct((v,), jnp.float32),
    )(t_t)


def _split_tc(indices, s_a, s_b):
    b, s = indices.shape
    rblk = 512
    assert b % rblk == 0

    def body(i_ref, a_ref, b_ref):
        x = i_ref[...]
        a_ref[...] = x[:, :s_a]
        b_ref[...] = jnp.concatenate(
            [x[:, s_a:s], jnp.zeros((rblk, LANE - s_b), jnp.int32)], axis=1)

    return pl.pallas_call(
        body,
        grid=(b // rblk,),
        in_specs=[pl.BlockSpec((rblk, s), lambda i: (i, 0))],
        out_specs=[pl.BlockSpec((rblk, LANE), lambda i: (i, 0)),
                   pl.BlockSpec((rblk, LANE), lambda i: (i, 0))],
        out_shape=[jax.ShapeDtypeStruct((b, LANE), jnp.int32),
                   jax.ShapeDtypeStruct((b, LANE), jnp.int32)],
    )(indices)


def _gather_sc(idx_a, idx_b, rsflat, s_b):
    n_rows = idx_a.shape[0]
    assert n_rows % NW == 0
    n_ch = n_rows // NW

    mesh = plsc.VectorSubcoreMesh(core_axis_name="c", subcore_axis_name="s")

    @functools.partial(
        pl.kernel,
        mesh=mesh,
        out_type=[jax.ShapeDtypeStruct((n_rows, LANE), jnp.float32),
                  jax.ShapeDtypeStruct((n_rows, LANE), jnp.float32)],
        scratch_types=[
            pltpu.VMEM((n_ch, LANE), jnp.int32),
            pltpu.VMEM((n_ch, LANE), jnp.int32),
            pltpu.VMEM((n_ch, LANE), jnp.float32),
            pltpu.VMEM((n_ch, LANE), jnp.float32),
            pltpu.SemaphoreType.DMA,
        ],
    )
    def k(ia_hbm, ib_hbm, rs_hbm, oa_hbm, ob_hbm, ia_v, ib_v, va_v, vb_v, sem):
        wid = lax.axis_index("s") * NC + lax.axis_index("c")
        row0 = wid * n_ch
        pltpu.sync_copy(ia_hbm.at[pl.ds(row0, n_ch)], ia_v)
        pltpu.sync_copy(ib_hbm.at[pl.ds(row0, n_ch)], ib_v)

        def start_a(c):
            pltpu.make_async_copy(
                rs_hbm.at[ia_v.at[c]], va_v.at[c], sem).start()

        def wait_a(c):
            pltpu.make_async_copy(
                rs_hbm.at[ia_v.at[c]], va_v.at[c], sem).wait()

        def start_b(c):
            pltpu.make_async_copy(
                rs_hbm.at[ib_v.at[c, pl.ds(0, s_b)]],
                vb_v.at[c, pl.ds(0, s_b)], sem).start()

        def wait_b(c):
            pltpu.make_async_copy(
                rs_hbm.at[ib_v.at[c, pl.ds(0, s_b)]],
                vb_v.at[c, pl.ds(0, s_b)], sem).wait()

        def ring(start, wait):
            assert (n_ch - RING) % UNROLL == 0

            def prime(c, carry):
                start(c)
                return carry
            lax.fori_loop(0, RING, prime, 0)

            def step(i, carry):
                c = i * UNROLL
                for u in range(UNROLL):
                    start(c + u + RING)
                    wait(c + u)
                return carry
            lax.fori_loop(0, (n_ch - RING) // UNROLL, step, 0)

            def drain(c, carry):
                wait(c)
                return carry
            lax.fori_loop(n_ch - RING, n_ch, drain, 0)

        ring(start_a, wait_a)
        ring(start_b, wait_b)

        pltpu.sync_copy(va_v, oa_hbm.at[pl.ds(row0, n_ch)])
        pltpu.sync_copy(vb_v, ob_hbm.at[pl.ds(row0, n_ch)])

    return k(idx_a, idx_b, rsflat)


def _stitch_mul_tc(g_a, g_b, weights, s_a, s_b):
    b, s = weights.shape
    rblk = 512

    def body(a_ref, b_ref, w_ref, o_ref):
        g = jnp.concatenate([a_ref[...], b_ref[:, :s_b]], axis=1)
        o_ref[...] = g * w_ref[...]

    return pl.pallas_call(
        body,
        grid=(b // rblk,),
        in_specs=[
            pl.BlockSpec((rblk, LANE), lambda i: (i, 0)),
            pl.BlockSpec((rblk, LANE), lambda i: (i, 0)),
            pl.BlockSpec((rblk, s), lambda i: (i, 0)),
        ],
        out_specs=pl.BlockSpec((rblk, s), lambda i: (i, 0)),
        out_shape=jax.ShapeDtypeStruct((b, s), jnp.float32),
    )(g_a, g_b, weights)


def kernel(indices, weights, table):
    b, s, n = indices.shape
    assert n == 1 and b % NW == 0
    s_a = LANE
    s_b = s - s_a
    rsflat = _rowsum_tc(table)
    idx_a, idx_b = _split_tc(indices.reshape(b, s).astype(jnp.int32), s_a, s_b)
    g_a, g_b = _gather_sc(idx_a, idx_b, rsflat, s_b)
    return _stitch_mul_tc(g_a, g_b, weights.reshape(b, s), s_a, s_b)
